# baseline (speedup 1.0000x reference)
import functools
import math

import numpy as np
import jax
import jax.numpy as jnp
from jax import lax
from jax.experimental import pallas as pl
from jax.experimental.pallas import tpu as pltpu
from jax.experimental.pallas import tpu_sc as plsc

_MAX_WIDTH = 64
_STRIPES_NUM = 2
_FILL = 0.0


def _rotl(x, r):
    return ((x << np.uint32(r)) | (x >> np.uint32(32 - r))).astype(np.uint32)


def _threefry2x32_pair(k0, k1, x0, x1):
    x0 = np.asarray(x0, np.uint32).copy()
    x1 = np.asarray(x1, np.uint32).copy()
    ks = [np.uint32(k0), np.uint32(k1),
          np.uint32(np.uint32(k0) ^ np.uint32(k1) ^ np.uint32(0x1BD11BDA))]
    rotations = [(13, 15, 26, 6), (17, 29, 16, 24)]
    with np.errstate(over="ignore"):
        x0 = (x0 + ks[0]).astype(np.uint32)
        x1 = (x1 + ks[1]).astype(np.uint32)
        for i in range(5):
            for r in rotations[i % 2]:
                x0 = (x0 + x1).astype(np.uint32)
                x1 = _rotl(x1, r)
                x1 = x1 ^ x0
            x0 = (x0 + ks[(i + 1) % 3]).astype(np.uint32)
            x1 = (x1 + ks[(i + 2) % 3] + np.uint32(i + 1)).astype(np.uint32)
    return x0, x1


def _iota_2x32(n):
    i = np.arange(n, dtype=np.uint64)
    return ((i >> np.uint64(32)).astype(np.uint32),
            (i & np.uint64(0xFFFFFFFF)).astype(np.uint32))


def _np_split(key):
    c1, c2 = _iota_2x32(2)
    b1, b2 = _threefry2x32_pair(key[0], key[1], c1, c2)
    return (b1[0], b2[0]), (b1[1], b2[1])


def _np_random_bits(key, shape):
    n = int(np.prod(shape)) if shape else 1
    c1, c2 = _iota_2x32(n)
    b1, b2 = _threefry2x32_pair(key[0], key[1], c1, c2)
    out = b1 ^ b2
    return out.reshape(shape) if shape else out[0]


def _np_randint(key, shape, minval, maxval):
    k1, k2 = _np_split(key)
    hi = _np_random_bits(k1, shape).astype(np.uint32)
    lo = _np_random_bits(k2, shape).astype(np.uint32)
    span = np.uint32(maxval - minval) if maxval > minval else np.uint32(1)
    with np.errstate(over="ignore"):
        m = np.uint32((np.uint32(65536) % span) * (np.uint32(65536) % span)) % span
        r = (np.uint32(hi % span * m) + lo % span).astype(np.uint32) % span
    return (np.int32(minval) + r.astype(np.int32)).astype(np.int32)


@functools.lru_cache(maxsize=None)
def _stripes(total_width: int):
    mw = min(_MAX_WIDTH, total_width)
    key = (np.uint32(0), np.uint32(42))
    key, k1 = _np_split(key)
    widths = _np_randint(k1, (_STRIPES_NUM,), 0, mw)
    starts = []
    for i in range(_STRIPES_NUM):
        key, k = _np_split(key)
        starts.append(int(_np_randint(k, (), 0, total_width - int(widths[i]))))
    return tuple(int(w) for w in widths), tuple(starts)


@functools.lru_cache(maxsize=None)
def _segments(total_width: int):
    widths, starts = _stripes(total_width)
    ivs = sorted((s, s + w) for s, w in zip(starts, widths) if w > 0)
    merged = []
    for a, b in ivs:
        if merged and a <= merged[-1][1]:
            merged[-1][1] = max(merged[-1][1], b)
        else:
            merged.append([a, b])
    segs, p = [], 0
    for a, b in merged:
        if p < a:
            segs.append((p, a - p, False))
        segs.append((a, b - a, True))
        p = b
    if p < total_width:
        segs.append((p, total_width - p, False))
    return tuple(segs)


@functools.lru_cache(maxsize=None)
def _build_sc_kernel(B: int, T: int, F: int):
    segs = _segments(T)
    masked = [(s, n) for s, n, m in segs if m]
    kept = [(s, n) for s, n, m in segs if not m]
    wmax = max([n for _, n in masked], default=1)

    info = plsc.get_sparse_core_info()
    NC, NS = info.num_cores, info.num_subcores
    NW = NC * NS
    jobs = math.ceil(B / NW)
    mesh = plsc.VectorSubcoreMesh(core_axis_name="c", subcore_axis_name="s")

    CH = 15360
    SLOTS = 8
    chunks = []
    for s, n in kept:
        off, rem = s * F, n * F
        while rem > 0:
            ln = min(CH, rem)
            chunks.append((off, ln))
            off += ln
            rem -= ln
    nch = len(chunks)

    @functools.partial(
        pl.kernel,
        mesh=mesh,
        out_type=jax.ShapeDtypeStruct((B * T * F,), jnp.float32),
        scratch_types=(
            [pltpu.VMEM((CH,), jnp.float32) for _ in range(SLOTS)] + [
                pltpu.VMEM((max(wmax * F, 16),), jnp.float32),
                pltpu.SemaphoreType.DMA,
                pltpu.SemaphoreType.DMA,
                pltpu.SemaphoreType.DMA,
            ]
        ),
    )
    def k(x_hbm, out_hbm, *rest):
        bufs = rest[:SLOTS]
        zbuf, isem, osem, zsem = rest[SLOTS:]
        wid = lax.axis_index("s") * NC + lax.axis_index("c")
        if masked:
            zv = jnp.zeros((16,), jnp.float32)

            def _zfill(c, carry):
                zbuf[pl.ds(c * 16, 16)] = zv
                return carry

            lax.fori_loop(0, max(wmax * F, 16) // 16, _zfill, 0)
        for j in range(jobs):
            b = wid + j * NW
            base = b * (T * F)

            def _issue(base):
                zh = [pltpu.async_copy(
                          zbuf.at[pl.ds(0, n * F)],
                          out_hbm.at[pl.ds(base + s * F, n * F)], zsem)
                      for s, n in masked]
                in_h = [None] * nch
                out_h = [None] * nch

                def start_in(i):
                    off, ln = chunks[i]
                    return pltpu.async_copy(
                        x_hbm.at[pl.ds(base + off, ln)],
                        bufs[i % SLOTS].at[pl.ds(0, ln)], isem)

                def start_out(i):
                    off, ln = chunks[i]
                    return pltpu.async_copy(
                        bufs[i % SLOTS].at[pl.ds(0, ln)],
                        out_hbm.at[pl.ds(base + off, ln)], osem)

                for i in range(nch):
                    if i >= SLOTS:
                        out_h[i - SLOTS].wait()
                    in_h[i] = start_in(i)
                    jj = i - (SLOTS - 1)
                    if jj >= 0:
                        in_h[jj].wait()
                        out_h[jj] = start_out(jj)
                for jj in range(max(0, nch - (SLOTS - 1)), nch):
                    in_h[jj].wait()
                    out_h[jj] = start_out(jj)
                for jj in range(max(0, nch - SLOTS), nch):
                    out_h[jj].wait()
                for h in zh:
                    h.wait()

            if B % NW == 0:
                _issue(base)
            else:
                @pl.when(b < B)
                def _():
                    _issue(base)

    return k


def kernel(x):
    B, T, F = x.shape
    return _build_sc_kernel(B, T, F)(x.reshape(-1)).reshape(B, T, F)


# speedup vs baseline: 1.0233x; 1.0233x over previous
import functools
import math

import numpy as np
import jax
import jax.numpy as jnp
from jax import lax
from jax.experimental import pallas as pl
from jax.experimental.pallas import tpu as pltpu
from jax.experimental.pallas import tpu_sc as plsc

_MAX_WIDTH = 64
_STRIPES_NUM = 2
_FILL = 0.0


def _rotl(x, r):
    return ((x << np.uint32(r)) | (x >> np.uint32(32 - r))).astype(np.uint32)


def _threefry2x32_pair(k0, k1, x0, x1):
    x0 = np.asarray(x0, np.uint32).copy()
    x1 = np.asarray(x1, np.uint32).copy()
    ks = [np.uint32(k0), np.uint32(k1),
          np.uint32(np.uint32(k0) ^ np.uint32(k1) ^ np.uint32(0x1BD11BDA))]
    rotations = [(13, 15, 26, 6), (17, 29, 16, 24)]
    with np.errstate(over="ignore"):
        x0 = (x0 + ks[0]).astype(np.uint32)
        x1 = (x1 + ks[1]).astype(np.uint32)
        for i in range(5):
            for r in rotations[i % 2]:
                x0 = (x0 + x1).astype(np.uint32)
                x1 = _rotl(x1, r)
                x1 = x1 ^ x0
            x0 = (x0 + ks[(i + 1) % 3]).astype(np.uint32)
            x1 = (x1 + ks[(i + 2) % 3] + np.uint32(i + 1)).astype(np.uint32)
    return x0, x1


def _iota_2x32(n):
    i = np.arange(n, dtype=np.uint64)
    return ((i >> np.uint64(32)).astype(np.uint32),
            (i & np.uint64(0xFFFFFFFF)).astype(np.uint32))


def _np_split(key):
    c1, c2 = _iota_2x32(2)
    b1, b2 = _threefry2x32_pair(key[0], key[1], c1, c2)
    return (b1[0], b2[0]), (b1[1], b2[1])


def _np_random_bits(key, shape):
    n = int(np.prod(shape)) if shape else 1
    c1, c2 = _iota_2x32(n)
    b1, b2 = _threefry2x32_pair(key[0], key[1], c1, c2)
    out = b1 ^ b2
    return out.reshape(shape) if shape else out[0]


def _np_randint(key, shape, minval, maxval):
    k1, k2 = _np_split(key)
    hi = _np_random_bits(k1, shape).astype(np.uint32)
    lo = _np_random_bits(k2, shape).astype(np.uint32)
    span = np.uint32(maxval - minval) if maxval > minval else np.uint32(1)
    with np.errstate(over="ignore"):
        m = np.uint32((np.uint32(65536) % span) * (np.uint32(65536) % span)) % span
        r = (np.uint32(hi % span * m) + lo % span).astype(np.uint32) % span
    return (np.int32(minval) + r.astype(np.int32)).astype(np.int32)


@functools.lru_cache(maxsize=None)
def _stripes(total_width: int):
    mw = min(_MAX_WIDTH, total_width)
    key = (np.uint32(0), np.uint32(42))
    key, k1 = _np_split(key)
    widths = _np_randint(k1, (_STRIPES_NUM,), 0, mw)
    starts = []
    for i in range(_STRIPES_NUM):
        key, k = _np_split(key)
        starts.append(int(_np_randint(k, (), 0, total_width - int(widths[i]))))
    return tuple(int(w) for w in widths), tuple(starts)


@functools.lru_cache(maxsize=None)
def _segments(total_width: int):
    widths, starts = _stripes(total_width)
    ivs = sorted((s, s + w) for s, w in zip(starts, widths) if w > 0)
    merged = []
    for a, b in ivs:
        if merged and a <= merged[-1][1]:
            merged[-1][1] = max(merged[-1][1], b)
        else:
            merged.append([a, b])
    segs, p = [], 0
    for a, b in merged:
        if p < a:
            segs.append((p, a - p, False))
        segs.append((a, b - a, True))
        p = b
    if p < total_width:
        segs.append((p, total_width - p, False))
    return tuple(segs)


@functools.lru_cache(maxsize=None)
def _build_sc_kernel(B: int, T: int, F: int):
    segs = _segments(T)
    masked = [(s, n) for s, n, m in segs if m]
    kept = [(s, n) for s, n, m in segs if not m]
    wmax = max([n for _, n in masked], default=1)

    info = plsc.get_sparse_core_info()
    NC, NS = info.num_cores, info.num_subcores
    NW = NC * NS
    jobs = math.ceil(B / NW)
    mesh = plsc.VectorSubcoreMesh(core_axis_name="c", subcore_axis_name="s")

    CH = 32768
    SLOTS = 3
    chunks = []
    for s, n in kept:
        off, rem = s * F, n * F
        while rem > 0:
            ln = min(CH, rem)
            chunks.append((off, ln))
            off += ln
            rem -= ln
    nch = len(chunks)

    @functools.partial(
        pl.kernel,
        mesh=mesh,
        out_type=jax.ShapeDtypeStruct((B * T * F,), jnp.float32),
        scratch_types=(
            [pltpu.VMEM((CH,), jnp.float32) for _ in range(SLOTS)] + [
                pltpu.VMEM((max(wmax * F, 16),), jnp.float32),
                pltpu.SemaphoreType.DMA,
                pltpu.SemaphoreType.DMA,
                pltpu.SemaphoreType.DMA,
            ]
        ),
    )
    def k(x_hbm, out_hbm, *rest):
        bufs = rest[:SLOTS]
        zbuf, isem, osem, zsem = rest[SLOTS:]
        wid = lax.axis_index("s") * NC + lax.axis_index("c")
        if masked:
            zv = jnp.zeros((16,), jnp.float32)

            def _zfill(c, carry):
                zbuf[pl.ds(c * 16, 16)] = zv
                return carry

            lax.fori_loop(0, max(wmax * F, 16) // 16, _zfill, 0)
        for j in range(jobs):
            b = wid + j * NW
            base = b * (T * F)

            def _issue(base):
                zh = [pltpu.async_copy(
                          zbuf.at[pl.ds(0, n * F)],
                          out_hbm.at[pl.ds(base + s * F, n * F)], zsem)
                      for s, n in masked]
                in_h = [None] * nch
                out_h = [None] * nch

                def start_in(i):
                    off, ln = chunks[i]
                    return pltpu.async_copy(
                        x_hbm.at[pl.ds(base + off, ln)],
                        bufs[i % SLOTS].at[pl.ds(0, ln)], isem)

                def start_out(i):
                    off, ln = chunks[i]
                    return pltpu.async_copy(
                        bufs[i % SLOTS].at[pl.ds(0, ln)],
                        out_hbm.at[pl.ds(base + off, ln)], osem)

                for i in range(nch):
                    if i >= SLOTS:
                        out_h[i - SLOTS].wait()
                    in_h[i] = start_in(i)
                    jj = i - (SLOTS - 1)
                    if jj >= 0:
                        in_h[jj].wait()
                        out_h[jj] = start_out(jj)
                for jj in range(max(0, nch - (SLOTS - 1)), nch):
                    in_h[jj].wait()
                    out_h[jj] = start_out(jj)
                for jj in range(max(0, nch - SLOTS), nch):
                    out_h[jj].wait()
                for h in zh:
                    h.wait()

            if B % NW == 0:
                _issue(base)
            else:
                @pl.when(b < B)
                def _():
                    _issue(base)

    return k


def kernel(x):
    B, T, F = x.shape
    return _build_sc_kernel(B, T, F)(x.reshape(-1)).reshape(B, T, F)
